# baseline (speedup 1.0000x reference)
import functools

import jax
import jax.numpy as jnp
from jax import lax
from jax.experimental import pallas as pl
from jax.experimental.pallas import tpu as pltpu
from jax.experimental.pallas import tpu_sc as plsc

NSEG = 128
NC = 2
NS = 16
NW = NC * NS
SEGW = NSEG // NW
CHUNK = 192
STPAD = 160
DEAD = 159
SCANW = 3136


def _pool_sc(h, bpad):
    n, hid = h.shape
    fch = hid // 16
    half_f = fch // 2

    mesh = plsc.VectorSubcoreMesh(
        core_axis_name="c", subcore_axis_name="s",
        num_cores=NC, num_subcores=NS)

    @functools.partial(
        pl.kernel,
        out_type=jax.ShapeDtypeStruct((NW, SEGW, 3 * hid), jnp.float32),
        mesh=mesh,
        scratch_types=[
            pltpu.VMEM((CHUNK, hid), jnp.float32),
            pltpu.VMEM((CHUNK, hid), jnp.float32),
            pltpu.VMEM((STPAD,), jnp.int32),
            pltpu.VMEM((SEGW * hid,), jnp.float32),
            pltpu.VMEM((SEGW * hid,), jnp.float32),
            pltpu.VMEM((SEGW * hid,), jnp.float32),
            pltpu.VMEM((SEGW, 3 * hid), jnp.float32),
            pltpu.VMEM((8 + SCANW,), jnp.int32),
            pltpu.VMEM((16,), jnp.int32),
            pltpu.VMEM_SHARED((STPAD,), jnp.int32),
            pltpu.SemaphoreType.DMA,
            pltpu.SemaphoreType.DMA,
        ],
    )
    def k(h_hbm, bp_hbm, g3_hbm, buf0, buf1, st_v,
          acc_s, acc_q, acc_m, outb, bscan, valbuf, stsh, sem0, sem1):
        sid = lax.axis_index("s")
        wid = lax.axis_index("c") * NS + sid
        s0 = wid * SEGW
        iota16 = lax.iota(jnp.int32, 16)

        zeros_i = jnp.zeros((16,), jnp.int32)

        @pl.when(sid == 0)
        def _():
            for c in range(STPAD // 16):
                bscan[pl.ds(c * 16, 16)] = zeros_i
            pltpu.sync_copy(bscan.at[pl.ds(0, STPAD)], stsh)

        lo_t = ((sid * n // NS) // 8) * 8
        hi_t = (((sid + 1) * n // NS) // 8) * 8
        pltpu.sync_copy(bp_hbm.at[pl.ds(lo_t, 8 + SCANW)], bscan)
        plsc.subcore_barrier()

        def scan_body(g, carry):
            o = 8 + g * 16
            ids = bscan[pl.ds(o, 16)]
            prev = bscan[pl.ds(o - 1, 16)]
            rowv = lo_t + g * 16 + iota16
            isb = (ids != prev) & (rowv < hi_t)

            @pl.when(prev[0] != ids[15])
            def _():
                valbuf[pl.ds(0, 16)] = jnp.where(isb, rowv + 1, 0)
                idxv = jnp.where(isb, ids, DEAD)
                pltpu.sync_copy(valbuf, stsh.at[idxv], add=True)
            return carry

        lax.fori_loop(0, SCANW // 16, scan_body, 0)
        plsc.subcore_barrier()
        pltpu.sync_copy(stsh, st_v)

        big = jnp.float32(9.0e7)
        carry = jnp.full((16,), jnp.float32(n + 1))
        for c in range(8, -1, -1):
            raw = st_v[pl.ds(c * 16, 16)]
            enc = jnp.where(raw == 0, big, raw.astype(jnp.float32))
            v = enc
            for sh in (1, 2, 4, 8):
                idxs = jnp.minimum(iota16 + sh, 15)
                v = jnp.minimum(v, v.at[idxs].get(mode="promise_in_bounds"))
            v = jnp.minimum(v, carry)
            st_v[pl.ds(c * 16, 16)] = v.astype(jnp.int32) - 1
            carry = jnp.full((16,), v[0])

        zeros = jnp.zeros((16,), jnp.float32)
        ninf = jnp.full((16,), -jnp.inf, jnp.float32)
        for j in range(SEGW):
            for f in range(fch):
                acc_s[pl.ds(j * hid + f * 16, 16)] = zeros
                acc_q[pl.ds(j * hid + f * 16, 16)] = zeros
                acc_m[pl.ds(j * hid + f * 16, 16)] = ninf

        def sload(idx):
            return st_v[pl.ds(idx, 16)][0]

        bufs = ((buf0, sem0), (buf1, sem1))

        for j in range(SEGW):
            a = sload(s0 + j)
            e = sload(s0 + j + 1)
            a8 = (a // 8) * 8
            nch = (e - a8 + (CHUNK - 1)) // CHUNK

            def cbase(kk, a8=a8):
                return jnp.minimum(a8 + kk * CHUNK, n - CHUNK)

            def start_dma(kk, bf, sm):
                pltpu.async_copy(h_hbm.at[pl.ds(cbase(kk), CHUNK)], bf, sm)

            def wait_dma(kk, bf, sm):
                pltpu.make_async_copy(
                    h_hbm.at[pl.ds(cbase(kk), CHUNK)], bf, sm).wait()

            def process(kk, bf, j=j, a=a, e=e, a8=a8):
                r0 = a8 + kk * CHUNK
                base = cbase(kk)
                lo = jnp.maximum(r0, a) - base
                hi = jnp.minimum(e - base, CHUNK)
                npairs = (hi - lo) // 2
                odd = ((hi - lo) & 1) == 1
                for half in range(2):
                    f0 = half * half_f

                    def pair_body(p, car, f0=f0, bf=bf, lo=lo):
                        r = lo + 2 * p
                        ns_ = list(car[0])
                        nq_ = list(car[1])
                        nm_ = list(car[2])
                        for rr in range(2):
                            for f in range(half_f):
                                v = bf[r + rr, pl.ds((f0 + f) * 16, 16)]
                                ns_[f] = ns_[f] + v
                                nq_[f] = nq_[f] + v * v
                                nm_[f] = jnp.maximum(nm_[f], v)
                        return (tuple(ns_), tuple(nq_), tuple(nm_))

                    init = (tuple(zeros for _ in range(half_f)),
                            tuple(zeros for _ in range(half_f)),
                            tuple(ninf for _ in range(half_f)))
                    ss, qq, mm = lax.fori_loop(0, npairs, pair_body, init)
                    for f in range(half_f):
                        o = j * hid + (f0 + f) * 16
                        vt = bf[hi - 1, pl.ds((f0 + f) * 16, 16)]
                        vz = jnp.where(odd, vt, 0.0)
                        sfin = ss[f] + vz
                        qfin = qq[f] + vz * vz
                        mfin = jnp.maximum(
                            mm[f], jnp.where(odd, vt, -jnp.inf))
                        acc_s[pl.ds(o, 16)] = acc_s[pl.ds(o, 16)] + sfin
                        acc_q[pl.ds(o, 16)] = acc_q[pl.ds(o, 16)] + qfin
                        acc_m[pl.ds(o, 16)] = jnp.maximum(
                            acc_m[pl.ds(o, 16)], mfin)

            @pl.when(nch > 0)
            def _():
                start_dma(0, buf0, sem0)

            def pair_body(g, carry, nch=nch):
                for b in range(2):
                    bf, sm = bufs[b]
                    obf, osm = bufs[1 - b]
                    k = 2 * g + b

                    @pl.when(k < nch)
                    def _(k=k, bf=bf, sm=sm, obf=obf, osm=osm):
                        wait_dma(k, bf, sm)

                        @pl.when(k + 1 < nch)
                        def _():
                            start_dma(k + 1, obf, osm)

                        process(k, bf)
                return carry

            lax.fori_loop(0, (nch + 1) // 2, pair_body, 0)

            cntf = (e - a).astype(jnp.float32)
            inv = jnp.ones((16,), jnp.float32) / jnp.full(
                (16,), jnp.maximum(cntf, 1.0), jnp.float32)
            for f in range(fch):
                o = j * hid + f * 16
                s_ = acc_s[pl.ds(o, 16)]
                q_ = acc_q[pl.ds(o, 16)]
                m_ = acc_m[pl.ds(o, 16)]
                mean = s_ * inv
                var = jnp.maximum(q_ * inv - mean * mean, 0.0)
                outb[j, pl.ds(f * 16, 16)] = mean
                outb[j, pl.ds(hid + f * 16, 16)] = m_
                outb[j, pl.ds(2 * hid + f * 16, 16)] = var

        pltpu.sync_copy(outb, g3_hbm.at[wid])

    return k(h, bpad).reshape(NSEG, 3 * hid)


def _mlp_tc(g3, W1, b1, W2, b2):
    nseg = g3.shape[0]
    hid = g3.shape[1] // 3
    nq = W2.shape[1]

    def body(g3_ref, w1_ref, b1_ref, w2_ref, b2_ref, z_ref):
        g3v = g3_ref[...]
        std = jnp.sqrt(g3v[:, 2 * hid:] + 1e-8)
        g = jnp.concatenate([g3v[:, :2 * hid], std], axis=1)
        hdn = jnp.maximum(
            jnp.dot(g, w1_ref[...], preferred_element_type=jnp.float32)
            + b1_ref[...], 0.0)
        z = jnp.tanh(
            jnp.dot(hdn, w2_ref[...], preferred_element_type=jnp.float32)
            + b2_ref[...])
        z_ref[...] = z * jnp.float32(jnp.pi)

    return pl.pallas_call(
        body,
        out_shape=jax.ShapeDtypeStruct((nseg, nq), jnp.float32),
    )(g3, W1, b1.reshape(1, -1), W2, b2.reshape(1, -1))


def kernel(h, W1, b1, W2, b2, batch):
    n = h.shape[0]
    b32 = batch.astype(jnp.int32)
    max_lo = (((NS - 1) * n // NS) // 8) * 8
    pad_len = max_lo + 8 + SCANW
    bpad = jnp.concatenate([
        jnp.full((8,), -1, jnp.int32),
        b32,
        jnp.zeros((pad_len - 8 - n,), jnp.int32)])
    g3 = _pool_sc(h, bpad)
    return _mlp_tc(g3, W1, b1, W2, b2)


# speedup vs baseline: 1.0870x; 1.0870x over previous
import functools

import jax
import jax.numpy as jnp
from jax import lax
from jax.experimental import pallas as pl
from jax.experimental.pallas import tpu as pltpu
from jax.experimental.pallas import tpu_sc as plsc

NSEG = 128
NC = 2
NS = 16
NW = NC * NS
SEGW = NSEG // NW
CHUNK = 192
STPAD = 160
DEAD = 159
SCANW = 3136


def _pool_sc(h, bpad):
    n, hid = h.shape
    fch = hid // 16
    half_f = fch // 2

    mesh = plsc.VectorSubcoreMesh(
        core_axis_name="c", subcore_axis_name="s",
        num_cores=NC, num_subcores=NS)

    @functools.partial(
        pl.kernel,
        out_type=jax.ShapeDtypeStruct((NW, SEGW, 3 * hid), jnp.float32),
        mesh=mesh,
        scratch_types=[
            pltpu.VMEM((CHUNK, hid), jnp.float32),
            pltpu.VMEM((CHUNK, hid), jnp.float32),
            pltpu.VMEM((STPAD,), jnp.int32),
            pltpu.VMEM((SEGW * hid,), jnp.float32),
            pltpu.VMEM((SEGW * hid,), jnp.float32),
            pltpu.VMEM((SEGW * hid,), jnp.float32),
            pltpu.VMEM((SEGW, 3 * hid), jnp.float32),
            pltpu.VMEM((16 + SCANW,), jnp.int32),
            pltpu.VMEM((16,), jnp.int32),
            pltpu.VMEM_SHARED((STPAD,), jnp.int32),
            pltpu.SemaphoreType.DMA,
            pltpu.SemaphoreType.DMA,
        ],
    )
    def k(h_hbm, bp_hbm, g3_hbm, buf0, buf1, st_v,
          acc_s, acc_q, acc_m, outb, bscan, valbuf, stsh, sem0, sem1):
        sid = lax.axis_index("s")
        wid = lax.axis_index("c") * NS + sid
        s0 = wid * SEGW
        iota16 = lax.iota(jnp.int32, 16)

        zeros_i = jnp.zeros((16,), jnp.int32)

        @pl.when(sid == 0)
        def _():
            for c in range(STPAD // 16):
                bscan[pl.ds(c * 16, 16)] = zeros_i
            pltpu.sync_copy(bscan.at[pl.ds(0, STPAD)], stsh)

        lo_t = ((sid * n // NS) // 8) * 8
        hi_t = (((sid + 1) * n // NS) // 8) * 8
        src_base = jnp.minimum(lo_t - 8, n - (8 + SCANW))

        @pl.when(lo_t == 0)
        def _():
            bscan[pl.ds(0, 16)] = jnp.full((16,), -1, jnp.int32)
            pltpu.sync_copy(bp_hbm.at[pl.ds(0, 8 + SCANW)],
                            bscan.at[pl.ds(8, 8 + SCANW)])

        @pl.when(lo_t > 0)
        def _():
            pltpu.sync_copy(bp_hbm.at[pl.ds(src_base, 8 + SCANW)],
                            bscan.at[pl.ds(0, 8 + SCANW)])

        idx0 = jnp.where(lo_t == 0, 8, lo_t - src_base)
        plsc.subcore_barrier()

        def scan_body(g, carry):
            o = idx0 + g * 16
            ids = bscan[pl.ds(o, 16)]
            prev = bscan[pl.ds(o - 1, 16)]
            rowv = lo_t + g * 16 + iota16
            isb = (ids != prev) & (rowv < hi_t)

            @pl.when(prev[0] != ids[15])
            def _():
                valbuf[pl.ds(0, 16)] = jnp.where(isb, rowv + 1, 0)
                idxv = jnp.where(isb, ids, DEAD)
                pltpu.sync_copy(valbuf, stsh.at[idxv], add=True)
            return carry

        lax.fori_loop(0, SCANW // 16, scan_body, 0)
        plsc.subcore_barrier()
        pltpu.sync_copy(stsh, st_v)

        big = jnp.float32(9.0e7)
        carry = jnp.full((16,), jnp.float32(n + 1))
        for c in range(8, -1, -1):
            raw = st_v[pl.ds(c * 16, 16)]
            enc = jnp.where(raw == 0, big, raw.astype(jnp.float32))
            v = enc
            for sh in (1, 2, 4, 8):
                idxs = jnp.minimum(iota16 + sh, 15)
                v = jnp.minimum(v, v.at[idxs].get(mode="promise_in_bounds"))
            v = jnp.minimum(v, carry)
            st_v[pl.ds(c * 16, 16)] = v.astype(jnp.int32) - 1
            carry = jnp.full((16,), v[0])

        zeros = jnp.zeros((16,), jnp.float32)
        ninf = jnp.full((16,), -jnp.inf, jnp.float32)
        for j in range(SEGW):
            for f in range(fch):
                acc_s[pl.ds(j * hid + f * 16, 16)] = zeros
                acc_q[pl.ds(j * hid + f * 16, 16)] = zeros
                acc_m[pl.ds(j * hid + f * 16, 16)] = ninf

        def sload(idx):
            return st_v[pl.ds(idx, 16)][0]

        bufs = ((buf0, sem0), (buf1, sem1))

        for j in range(SEGW):
            a = sload(s0 + j)
            e = sload(s0 + j + 1)
            a8 = (a // 8) * 8
            nch = (e - a8 + (CHUNK - 1)) // CHUNK

            def cbase(kk, a8=a8):
                return jnp.minimum(a8 + kk * CHUNK, n - CHUNK)

            def start_dma(kk, bf, sm):
                pltpu.async_copy(h_hbm.at[pl.ds(cbase(kk), CHUNK)], bf, sm)

            def wait_dma(kk, bf, sm):
                pltpu.make_async_copy(
                    h_hbm.at[pl.ds(cbase(kk), CHUNK)], bf, sm).wait()

            def process(kk, bf, j=j, a=a, e=e, a8=a8):
                r0 = a8 + kk * CHUNK
                base = cbase(kk)
                lo = jnp.maximum(r0, a) - base
                hi = jnp.minimum(e - base, CHUNK)
                for half in range(2):
                    f0 = half * half_f

                    def row_body(i, car, f0=f0, bf=bf):
                        ns_, nq_, nm_ = [], [], []
                        for f in range(half_f):
                            v = bf[i, pl.ds((f0 + f) * 16, 16)]
                            ns_.append(car[0][f] + v)
                            nq_.append(car[1][f] + v * v)
                            nm_.append(jnp.maximum(car[2][f], v))
                        return (tuple(ns_), tuple(nq_), tuple(nm_))

                    init = (tuple(zeros for _ in range(half_f)),
                            tuple(zeros for _ in range(half_f)),
                            tuple(ninf for _ in range(half_f)))
                    ss, qq, mm = lax.fori_loop(lo, hi, row_body, init)
                    for f in range(half_f):
                        o = j * hid + (f0 + f) * 16
                        acc_s[pl.ds(o, 16)] = acc_s[pl.ds(o, 16)] + ss[f]
                        acc_q[pl.ds(o, 16)] = acc_q[pl.ds(o, 16)] + qq[f]
                        acc_m[pl.ds(o, 16)] = jnp.maximum(
                            acc_m[pl.ds(o, 16)], mm[f])

            @pl.when(nch > 0)
            def _():
                start_dma(0, buf0, sem0)

            def pair_body(g, carry, nch=nch):
                for b in range(2):
                    bf, sm = bufs[b]
                    obf, osm = bufs[1 - b]
                    k = 2 * g + b

                    @pl.when(k < nch)
                    def _(k=k, bf=bf, sm=sm, obf=obf, osm=osm):
                        wait_dma(k, bf, sm)

                        @pl.when(k + 1 < nch)
                        def _():
                            start_dma(k + 1, obf, osm)

                        process(k, bf)
                return carry

            lax.fori_loop(0, (nch + 1) // 2, pair_body, 0)

            cntf = (e - a).astype(jnp.float32)
            inv = jnp.ones((16,), jnp.float32) / jnp.full(
                (16,), jnp.maximum(cntf, 1.0), jnp.float32)
            for f in range(fch):
                o = j * hid + f * 16
                s_ = acc_s[pl.ds(o, 16)]
                q_ = acc_q[pl.ds(o, 16)]
                m_ = acc_m[pl.ds(o, 16)]
                mean = s_ * inv
                var = jnp.maximum(q_ * inv - mean * mean, 0.0)
                outb[j, pl.ds(f * 16, 16)] = mean
                outb[j, pl.ds(hid + f * 16, 16)] = m_
                outb[j, pl.ds(2 * hid + f * 16, 16)] = var

        pltpu.sync_copy(outb, g3_hbm.at[wid])

    return k(h, bpad)


def _mlp_tc(g4, W1, b1, W2, b2):
    nseg = g4.shape[0] * g4.shape[1]
    hid = g4.shape[2] // 3
    nq = W2.shape[1]

    def body(g3_ref, w1_ref, b1_ref, w2_ref, b2_ref, z_ref):
        g3v = g3_ref[...].reshape(nseg, 3 * hid)
        std = jnp.sqrt(g3v[:, 2 * hid:] + 1e-8)
        g = jnp.concatenate([g3v[:, :2 * hid], std], axis=1)
        hdn = jnp.maximum(
            jnp.dot(g, w1_ref[...], preferred_element_type=jnp.float32)
            + b1_ref[...], 0.0)
        z = jnp.tanh(
            jnp.dot(hdn, w2_ref[...], preferred_element_type=jnp.float32)
            + b2_ref[...])
        z_ref[...] = z * jnp.float32(jnp.pi)

    return pl.pallas_call(
        body,
        out_shape=jax.ShapeDtypeStruct((nseg, nq), jnp.float32),
    )(g4, W1, b1.reshape(1, -1), W2, b2.reshape(1, -1))


def kernel(h, W1, b1, W2, b2, batch):
    b32 = batch.astype(jnp.int32)
    g4 = _pool_sc(h, b32)
    return _mlp_tc(g4, W1, b1, W2, b2)


# speedup vs baseline: 1.0889x; 1.0018x over previous
import functools

import jax
import jax.numpy as jnp
from jax import lax
from jax.experimental import pallas as pl
from jax.experimental.pallas import tpu as pltpu
from jax.experimental.pallas import tpu_sc as plsc

NSEG = 128
NC = 2
NS = 16
NW = NC * NS
SEGW = NSEG // NW
CHUNK = 192
STPAD = 160
DEAD = 159
SCANW = 3136


def _pool_sc(h, bpad):
    n, hid = h.shape
    fch = hid // 16
    half_f = fch // 2

    mesh = plsc.VectorSubcoreMesh(
        core_axis_name="c", subcore_axis_name="s",
        num_cores=NC, num_subcores=NS)

    @functools.partial(
        pl.kernel,
        out_type=jax.ShapeDtypeStruct((NW, SEGW, 3 * hid), jnp.float32),
        mesh=mesh,
        scratch_types=[
            pltpu.VMEM((CHUNK, hid), jnp.float32),
            pltpu.VMEM((CHUNK, hid), jnp.float32),
            pltpu.VMEM((STPAD,), jnp.int32),
            pltpu.VMEM((SEGW * hid,), jnp.float32),
            pltpu.VMEM((SEGW * hid,), jnp.float32),
            pltpu.VMEM((SEGW * hid,), jnp.float32),
            pltpu.VMEM((SEGW, 3 * hid), jnp.float32),
            pltpu.VMEM((16 + SCANW,), jnp.int32),
            pltpu.VMEM((16,), jnp.int32),
            pltpu.VMEM_SHARED((STPAD,), jnp.int32),
            pltpu.SemaphoreType.DMA,
            pltpu.SemaphoreType.DMA,
        ],
    )
    def k(h_hbm, bp_hbm, g3_hbm, buf0, buf1, st_v,
          acc_s, acc_q, acc_m, outb, bscan, valbuf, stsh, sem0, sem1):
        sid = lax.axis_index("s")
        wid = lax.axis_index("c") * NS + sid
        s0 = wid * SEGW
        iota16 = lax.iota(jnp.int32, 16)

        zeros_i = jnp.zeros((16,), jnp.int32)

        @pl.when(sid == 0)
        def _():
            for c in range(STPAD // 16):
                bscan[pl.ds(c * 16, 16)] = zeros_i
            pltpu.sync_copy(bscan.at[pl.ds(0, STPAD)], stsh)

        lo_t = ((sid * n // NS) // 8) * 8
        hi_t = (((sid + 1) * n // NS) // 8) * 8
        src_base = jnp.minimum(lo_t - 8, n - (8 + SCANW))

        @pl.when(lo_t == 0)
        def _():
            bscan[pl.ds(0, 16)] = jnp.full((16,), -1, jnp.int32)
            pltpu.sync_copy(bp_hbm.at[pl.ds(0, 8 + SCANW)],
                            bscan.at[pl.ds(8, 8 + SCANW)])

        @pl.when(lo_t > 0)
        def _():
            pltpu.sync_copy(bp_hbm.at[pl.ds(src_base, 8 + SCANW)],
                            bscan.at[pl.ds(0, 8 + SCANW)])

        idx0 = jnp.where(lo_t == 0, 8, lo_t - src_base)
        plsc.subcore_barrier()

        def scan_body(g, carry):
            o = idx0 + g * 16
            ids = bscan[pl.ds(o, 16)]
            prev = bscan[pl.ds(o - 1, 16)]
            rowv = lo_t + g * 16 + iota16
            isb = (ids != prev) & (rowv < hi_t)

            @pl.when(prev[0] != ids[15])
            def _():
                valbuf[pl.ds(0, 16)] = jnp.where(isb, rowv + 1, 0)
                idxv = jnp.where(isb, ids, DEAD)
                pltpu.sync_copy(valbuf, stsh.at[idxv], add=True)
            return carry

        lax.fori_loop(0, SCANW // 16, scan_body, 0)
        plsc.subcore_barrier()
        pltpu.sync_copy(stsh, st_v)

        big = jnp.float32(9.0e7)
        carry = jnp.full((16,), jnp.float32(n + 1))
        for c in range(8, -1, -1):
            raw = st_v[pl.ds(c * 16, 16)]
            enc = jnp.where(raw == 0, big, raw.astype(jnp.float32))
            v = enc
            for sh in (1, 2, 4, 8):
                idxs = jnp.minimum(iota16 + sh, 15)
                v = jnp.minimum(v, v.at[idxs].get(mode="promise_in_bounds"))
            v = jnp.minimum(v, carry)
            st_v[pl.ds(c * 16, 16)] = v.astype(jnp.int32) - 1
            carry = jnp.full((16,), v[0])

        zeros = jnp.zeros((16,), jnp.float32)
        ninf = jnp.full((16,), -jnp.inf, jnp.float32)
        for j in range(SEGW):
            for f in range(fch):
                acc_s[pl.ds(j * hid + f * 16, 16)] = zeros
                acc_q[pl.ds(j * hid + f * 16, 16)] = zeros
                acc_m[pl.ds(j * hid + f * 16, 16)] = ninf

        def sload(idx):
            return st_v[pl.ds(idx, 16)][0]

        bufs = ((buf0, sem0), (buf1, sem1))

        for j in range(SEGW):
            a = sload(s0 + j)
            e = sload(s0 + j + 1)
            a8 = (a // 8) * 8
            nch = (e - a8 + (CHUNK - 1)) // CHUNK

            def cbase(kk, a8=a8):
                return jnp.minimum(a8 + kk * CHUNK, n - CHUNK)

            def start_dma(kk, bf, sm):
                pltpu.async_copy(h_hbm.at[pl.ds(cbase(kk), CHUNK)], bf, sm)

            def wait_dma(kk, bf, sm):
                pltpu.make_async_copy(
                    h_hbm.at[pl.ds(cbase(kk), CHUNK)], bf, sm).wait()

            def process(kk, bf, j=j, a=a, e=e, a8=a8):
                r0 = a8 + kk * CHUNK
                base = cbase(kk)
                lo = jnp.maximum(r0, a) - base
                hi = jnp.minimum(e - base, CHUNK)
                def row_body(i, car, bf=bf):
                    ns_, nq_, nm_ = [], [], []
                    for f in range(fch):
                        v = bf[i, pl.ds(f * 16, 16)]
                        ns_.append(car[0][f] + v)
                        nq_.append(car[1][f] + v * v)
                        nm_.append(jnp.maximum(car[2][f], v))
                    return (tuple(ns_), tuple(nq_), tuple(nm_))

                init = (tuple(zeros for _ in range(fch)),
                        tuple(zeros for _ in range(fch)),
                        tuple(ninf for _ in range(fch)))
                ss, qq, mm = lax.fori_loop(lo, hi, row_body, init)
                for f in range(fch):
                    o = j * hid + f * 16
                    acc_s[pl.ds(o, 16)] = acc_s[pl.ds(o, 16)] + ss[f]
                    acc_q[pl.ds(o, 16)] = acc_q[pl.ds(o, 16)] + qq[f]
                    acc_m[pl.ds(o, 16)] = jnp.maximum(
                        acc_m[pl.ds(o, 16)], mm[f])

            @pl.when(nch > 0)
            def _():
                start_dma(0, buf0, sem0)

            def pair_body(g, carry, nch=nch):
                for b in range(2):
                    bf, sm = bufs[b]
                    obf, osm = bufs[1 - b]
                    k = 2 * g + b

                    @pl.when(k < nch)
                    def _(k=k, bf=bf, sm=sm, obf=obf, osm=osm):
                        wait_dma(k, bf, sm)

                        @pl.when(k + 1 < nch)
                        def _():
                            start_dma(k + 1, obf, osm)

                        process(k, bf)
                return carry

            lax.fori_loop(0, (nch + 1) // 2, pair_body, 0)

            cntf = (e - a).astype(jnp.float32)
            inv = jnp.ones((16,), jnp.float32) / jnp.full(
                (16,), jnp.maximum(cntf, 1.0), jnp.float32)
            for f in range(fch):
                o = j * hid + f * 16
                s_ = acc_s[pl.ds(o, 16)]
                q_ = acc_q[pl.ds(o, 16)]
                m_ = acc_m[pl.ds(o, 16)]
                mean = s_ * inv
                var = jnp.maximum(q_ * inv - mean * mean, 0.0)
                outb[j, pl.ds(f * 16, 16)] = mean
                outb[j, pl.ds(hid + f * 16, 16)] = m_
                outb[j, pl.ds(2 * hid + f * 16, 16)] = var

        pltpu.sync_copy(outb, g3_hbm.at[wid])

    return k(h, bpad)


def _mlp_tc(g4, W1, b1, W2, b2):
    nseg = g4.shape[0] * g4.shape[1]
    hid = g4.shape[2] // 3
    nq = W2.shape[1]

    def body(g3_ref, w1_ref, b1_ref, w2_ref, b2_ref, z_ref):
        g3v = g3_ref[...].reshape(nseg, 3 * hid)
        std = jnp.sqrt(g3v[:, 2 * hid:] + 1e-8)
        g = jnp.concatenate([g3v[:, :2 * hid], std], axis=1)
        hdn = jnp.maximum(
            jnp.dot(g, w1_ref[...], preferred_element_type=jnp.float32)
            + b1_ref[...], 0.0)
        z = jnp.tanh(
            jnp.dot(hdn, w2_ref[...], preferred_element_type=jnp.float32)
            + b2_ref[...])
        z_ref[...] = z * jnp.float32(jnp.pi)

    return pl.pallas_call(
        body,
        out_shape=jax.ShapeDtypeStruct((nseg, nq), jnp.float32),
    )(g4, W1, b1.reshape(1, -1), W2, b2.reshape(1, -1))


def kernel(h, W1, b1, W2, b2, batch):
    b32 = batch.astype(jnp.int32)
    g4 = _pool_sc(h, b32)
    return _mlp_tc(g4, W1, b1, W2, b2)
